# baseline (speedup 1.0000x reference)
import jax
import jax.numpy as jnp
from jax.experimental import pallas as pl
from jax.experimental.pallas import tpu as pltpu

_BM = 2048
_BSMALL = 1024
_NBUF = 3
_NEDGE = 2


def _schedule(m):
    sched = []
    row = 0
    for _ in range(_NEDGE):
        sched.append((row, _BSMALL))
        row += _BSMALL
    end_edge = m - _NEDGE * _BSMALL
    while row < end_edge:
        sched.append((row, _BM))
        row += _BM
    for _ in range(_NEDGE):
        sched.append((row, _BSMALL))
        row += _BSMALL
    assert row == m
    return sched


def _x_copy(x_hbm, xbuf, xsem, off, n, slot):
    return pltpu.make_async_copy(
        x_hbm.at[pl.ds(off, n), :], xbuf.at[slot, pl.ds(0, n), :],
        xsem.at[slot])


def _o_copy(o_hbm, obuf, osem, off, n, slot):
    return pltpu.make_async_copy(
        obuf.at[slot, pl.ds(0, n), :], o_hbm.at[pl.ds(off, n), :],
        osem.at[slot])


def _affine_kernel(x_hbm, w_hbm, b_ref, o_hbm,
                   xbuf, obuf, wb, xsem, osem, wsem):
    sched = _schedule(x_hbm.shape[0])
    n_steps = len(sched)
    d = w_hbm.shape[0]

    off0, n0 = sched[0]
    _x_copy(x_hbm, xbuf, xsem, off0, n0, 0).start()
    w_dma = pltpu.make_async_copy(w_hbm, obuf.at[0, pl.ds(0, d), :], wsem)
    w_dma.start()
    for t in range(1, _NBUF):
        off, n = sched[t]
        _x_copy(x_hbm, xbuf, xsem, off, n, t).start()
    w_dma.wait()
    wb[...] = obuf[0, pl.ds(0, d), :].astype(jnp.bfloat16)
    bias = b_ref[...]

    for t, (off, n) in enumerate(sched):
        slot = t % _NBUF
        _x_copy(x_hbm, xbuf, xsem, off, n, slot).wait()
        xc = xbuf[slot, pl.ds(0, n), :].astype(jnp.bfloat16)
        if t + _NBUF < n_steps:
            noff, nn = sched[t + _NBUF]
            _x_copy(x_hbm, xbuf, xsem, noff, nn, slot).start()
        if t >= _NBUF:
            poff, pn = sched[t - _NBUF]
            _o_copy(o_hbm, obuf, osem, poff, pn, slot).wait()
        acc = jax.lax.dot_general(
            xc, wb[...],
            dimension_numbers=(((1,), (1,)), ((), ())),
            preferred_element_type=jnp.float32,
        )
        obuf[slot, pl.ds(0, n), :] = acc + bias
        _o_copy(o_hbm, obuf, osem, off, n, slot).start()

    for t in range(n_steps - _NBUF, n_steps):
        off, n = sched[t]
        _o_copy(o_hbm, obuf, osem, off, n, t % _NBUF).wait()


def kernel(flat, cu_seqlens, W, b):
    del cu_seqlens
    M, d = flat.shape
    return pl.pallas_call(
        _affine_kernel,
        in_specs=[
            pl.BlockSpec(memory_space=pltpu.MemorySpace.HBM),
            pl.BlockSpec(memory_space=pltpu.MemorySpace.HBM),
            pl.BlockSpec(memory_space=pltpu.MemorySpace.VMEM),
        ],
        out_specs=pl.BlockSpec(memory_space=pltpu.MemorySpace.HBM),
        out_shape=jax.ShapeDtypeStruct((M, d), jnp.float32),
        scratch_shapes=[
            pltpu.VMEM((_NBUF, _BM, d), jnp.float32),
            pltpu.VMEM((_NBUF, _BM, d), jnp.float32),
            pltpu.VMEM((d, d), jnp.bfloat16),
            pltpu.SemaphoreType.DMA((_NBUF,)),
            pltpu.SemaphoreType.DMA((_NBUF,)),
            pltpu.SemaphoreType.DMA,
        ],
    )(flat, W, b.reshape(1, d))


# speedup vs baseline: 1.0229x; 1.0229x over previous
import jax
import jax.numpy as jnp
from jax.experimental import pallas as pl
from jax.experimental.pallas import tpu as pltpu

_BM = 2048
_BSMALL = 1024
_NEDGE = 2
_NX = 4
_BU = 1024
_NO = 3


def _schedule(m):
    sched = []
    row = 0
    for _ in range(_NEDGE):
        sched.append((row, _BSMALL))
        row += _BSMALL
    end_edge = m - _NEDGE * _BSMALL
    while row < end_edge:
        sched.append((row, _BM))
        row += _BM
    for _ in range(_NEDGE):
        sched.append((row, _BSMALL))
        row += _BSMALL
    assert row == m
    return sched


def _x_copy(x_hbm, xbuf, xsem, off, n, slot):
    return pltpu.make_async_copy(
        x_hbm.at[pl.ds(off, n), :], xbuf.at[slot, pl.ds(0, n), :],
        xsem.at[slot])


def _o_copy(o_hbm, obuf, osem, off, slot):
    return pltpu.make_async_copy(
        obuf.at[slot], o_hbm.at[pl.ds(off, _BU), :], osem.at[slot])


def _affine_kernel(x_hbm, w_hbm, b_ref, o_hbm,
                   xbuf, obuf, wb, xsem, osem, wsem):
    sched = _schedule(x_hbm.shape[0])
    n_steps = len(sched)
    d = w_hbm.shape[0]

    off0, n0 = sched[0]
    _x_copy(x_hbm, xbuf, xsem, off0, n0, 0).start()
    w_dma = pltpu.make_async_copy(w_hbm, xbuf.at[_NX - 1, pl.ds(0, d), :],
                                  wsem)
    w_dma.start()
    for t in range(1, _NX - 1):
        off, n = sched[t]
        _x_copy(x_hbm, xbuf, xsem, off, n, t).start()
    w_dma.wait()
    wb[...] = xbuf[_NX - 1, pl.ds(0, d), :].astype(jnp.bfloat16)
    bias = b_ref[...]
    offw, nw = sched[_NX - 1]
    _x_copy(x_hbm, xbuf, xsem, offw, nw, _NX - 1).start()

    unit_offs = []
    for off, n in sched:
        for h in range(n // _BU):
            unit_offs.append(off + h * _BU)

    u = 0
    for t, (off, n) in enumerate(sched):
        slot = t % _NX
        _x_copy(x_hbm, xbuf, xsem, off, n, slot).wait()
        acc = jax.lax.dot_general(
            xbuf[slot, pl.ds(0, n), :].astype(jnp.bfloat16), wb[...],
            dimension_numbers=(((1,), (1,)), ((), ())),
            preferred_element_type=jnp.float32,
        )
        for h in range(n // _BU):
            uslot = u % _NO
            if u >= _NO:
                _o_copy(o_hbm, obuf, osem, unit_offs[u - _NO], uslot).wait()
            obuf[uslot] = acc[h * _BU:(h + 1) * _BU, :] + bias
            _o_copy(o_hbm, obuf, osem, unit_offs[u], uslot).start()
            u += 1
        if t + _NX < n_steps:
            noff, nn = sched[t + _NX]
            _x_copy(x_hbm, xbuf, xsem, noff, nn, slot).start()

    n_units = len(unit_offs)
    for uu in range(n_units - _NO, n_units):
        _o_copy(o_hbm, obuf, osem, unit_offs[uu], uu % _NO).wait()


def kernel(flat, cu_seqlens, W, b):
    del cu_seqlens
    M, d = flat.shape
    return pl.pallas_call(
        _affine_kernel,
        in_specs=[
            pl.BlockSpec(memory_space=pltpu.MemorySpace.HBM),
            pl.BlockSpec(memory_space=pltpu.MemorySpace.HBM),
            pl.BlockSpec(memory_space=pltpu.MemorySpace.VMEM),
        ],
        out_specs=pl.BlockSpec(memory_space=pltpu.MemorySpace.HBM),
        out_shape=jax.ShapeDtypeStruct((M, d), jnp.float32),
        scratch_shapes=[
            pltpu.VMEM((_NX, _BM, d), jnp.float32),
            pltpu.VMEM((_NO, _BU, d), jnp.float32),
            pltpu.VMEM((d, d), jnp.bfloat16),
            pltpu.SemaphoreType.DMA((_NX,)),
            pltpu.SemaphoreType.DMA((_NO,)),
            pltpu.SemaphoreType.DMA,
        ],
    )(flat, W, b.reshape(1, d))


# speedup vs baseline: 1.0808x; 1.0566x over previous
import jax
import jax.numpy as jnp
from jax.experimental import pallas as pl
from jax.experimental.pallas import tpu as pltpu

_BM = 2048
_BSMALL = 1536
_NBUF = 3
_NEDGE = 2


def _schedule(m):
    sched = []
    row = 0
    for _ in range(_NEDGE):
        sched.append((row, _BSMALL))
        row += _BSMALL
    end_edge = m - _NEDGE * _BSMALL
    while row < end_edge:
        sched.append((row, _BM))
        row += _BM
    for _ in range(_NEDGE):
        sched.append((row, _BSMALL))
        row += _BSMALL
    assert row == m
    return sched


def _x_copy(x_hbm, xbuf, xsem, off, n, slot):
    return pltpu.make_async_copy(
        x_hbm.at[pl.ds(off, n), :], xbuf.at[slot, pl.ds(0, n), :],
        xsem.at[slot])


def _o_copy(o_hbm, obuf, osem, off, n, slot):
    return pltpu.make_async_copy(
        obuf.at[slot, pl.ds(0, n), :], o_hbm.at[pl.ds(off, n), :],
        osem.at[slot])


def _affine_kernel(x_hbm, w_hbm, b_ref, o_hbm,
                   xbuf, obuf, wb, xsem, osem, wsem):
    sched = _schedule(x_hbm.shape[0])
    n_steps = len(sched)
    d = w_hbm.shape[0]

    off0, n0 = sched[0]
    _x_copy(x_hbm, xbuf, xsem, off0, n0, 0).start()
    w_dma = pltpu.make_async_copy(w_hbm, obuf.at[0, pl.ds(0, d), :], wsem)
    w_dma.start()
    for t in range(1, _NBUF):
        off, n = sched[t]
        _x_copy(x_hbm, xbuf, xsem, off, n, t).start()
    w_dma.wait()
    wb[...] = obuf[0, pl.ds(0, d), :].astype(jnp.bfloat16)
    bias = b_ref[...]

    for t, (off, n) in enumerate(sched):
        slot = t % _NBUF
        _x_copy(x_hbm, xbuf, xsem, off, n, slot).wait()
        if t >= _NBUF:
            poff, pn = sched[t - _NBUF]
            _o_copy(o_hbm, obuf, osem, poff, pn, slot).wait()
        acc = jax.lax.dot_general(
            xbuf[slot, pl.ds(0, n), :].astype(jnp.bfloat16), wb[...],
            dimension_numbers=(((1,), (1,)), ((), ())),
            preferred_element_type=jnp.float32,
        )
        obuf[slot, pl.ds(0, n), :] = acc + bias
        _o_copy(o_hbm, obuf, osem, off, n, slot).start()
        if t + _NBUF < n_steps:
            noff, nn = sched[t + _NBUF]
            _x_copy(x_hbm, xbuf, xsem, noff, nn, slot).start()

    for t in range(n_steps - _NBUF, n_steps):
        off, n = sched[t]
        _o_copy(o_hbm, obuf, osem, off, n, t % _NBUF).wait()


def kernel(flat, cu_seqlens, W, b):
    del cu_seqlens
    M, d = flat.shape
    return pl.pallas_call(
        _affine_kernel,
        in_specs=[
            pl.BlockSpec(memory_space=pltpu.MemorySpace.HBM),
            pl.BlockSpec(memory_space=pltpu.MemorySpace.HBM),
            pl.BlockSpec(memory_space=pltpu.MemorySpace.VMEM),
        ],
        out_specs=pl.BlockSpec(memory_space=pltpu.MemorySpace.HBM),
        out_shape=jax.ShapeDtypeStruct((M, d), jnp.float32),
        scratch_shapes=[
            pltpu.VMEM((_NBUF, _BM, d), jnp.float32),
            pltpu.VMEM((_NBUF, _BM, d), jnp.float32),
            pltpu.VMEM((d, d), jnp.bfloat16),
            pltpu.SemaphoreType.DMA((_NBUF,)),
            pltpu.SemaphoreType.DMA((_NBUF,)),
            pltpu.SemaphoreType.DMA,
        ],
    )(flat, W, b.reshape(1, d))


# speedup vs baseline: 1.0922x; 1.0106x over previous
import jax
import jax.numpy as jnp
from jax.experimental import pallas as pl
from jax.experimental.pallas import tpu as pltpu

_BM = 2048
_BSMALL = 1024
_NBUF = 3
_NEDGE = 2


def _schedule(m):
    sched = []
    row = 0
    for _ in range(_NEDGE):
        sched.append((row, _BSMALL))
        row += _BSMALL
    end_edge = m - _NEDGE * _BSMALL
    while row < end_edge:
        sched.append((row, _BM))
        row += _BM
    for _ in range(_NEDGE):
        sched.append((row, _BSMALL))
        row += _BSMALL
    assert row == m
    return sched


def _x_copy(x_hbm, xbuf, xsem, off, n, slot):
    return pltpu.make_async_copy(
        x_hbm.at[pl.ds(off, n), :], xbuf.at[slot, pl.ds(0, n), :],
        xsem.at[slot])


def _o_copy(o_hbm, obuf, osem, off, n, slot):
    return pltpu.make_async_copy(
        obuf.at[slot, pl.ds(0, n), :], o_hbm.at[pl.ds(off, n), :],
        osem.at[slot])


def _affine_kernel(x_hbm, w_hbm, b_ref, o_hbm,
                   xbuf, obuf, wb, xsem, osem, wsem):
    sched = _schedule(x_hbm.shape[0])
    n_steps = len(sched)
    d = w_hbm.shape[0]

    off0, n0 = sched[0]
    _x_copy(x_hbm, xbuf, xsem, off0, n0, 0).start()
    w_dma = pltpu.make_async_copy(w_hbm, obuf.at[0, pl.ds(0, d), :], wsem)
    w_dma.start()
    for t in range(1, _NBUF):
        off, n = sched[t]
        _x_copy(x_hbm, xbuf, xsem, off, n, t).start()
    w_dma.wait()
    wb[...] = obuf[0, pl.ds(0, d), :].astype(jnp.bfloat16)
    bias = b_ref[...]

    for t, (off, n) in enumerate(sched):
        slot = t % _NBUF
        _x_copy(x_hbm, xbuf, xsem, off, n, slot).wait()
        if t >= _NBUF:
            poff, pn = sched[t - _NBUF]
            _o_copy(o_hbm, obuf, osem, poff, pn, slot).wait()
        acc = jax.lax.dot_general(
            xbuf[slot, pl.ds(0, n), :].astype(jnp.bfloat16), wb[...],
            dimension_numbers=(((1,), (1,)), ((), ())),
            preferred_element_type=jnp.float32,
        )
        obuf[slot, pl.ds(0, n), :] = acc + bias
        _o_copy(o_hbm, obuf, osem, off, n, slot).start()
        if t + _NBUF < n_steps:
            noff, nn = sched[t + _NBUF]
            _x_copy(x_hbm, xbuf, xsem, noff, nn, slot).start()

    for t in range(n_steps - _NBUF, n_steps):
        off, n = sched[t]
        _o_copy(o_hbm, obuf, osem, off, n, t % _NBUF).wait()


def kernel(flat, cu_seqlens, W, b):
    del cu_seqlens
    M, d = flat.shape
    return pl.pallas_call(
        _affine_kernel,
        in_specs=[
            pl.BlockSpec(memory_space=pltpu.MemorySpace.HBM),
            pl.BlockSpec(memory_space=pltpu.MemorySpace.HBM),
            pl.BlockSpec(memory_space=pltpu.MemorySpace.VMEM),
        ],
        out_specs=pl.BlockSpec(memory_space=pltpu.MemorySpace.HBM),
        out_shape=jax.ShapeDtypeStruct((M, d), jnp.float32),
        scratch_shapes=[
            pltpu.VMEM((_NBUF, _BM, d), jnp.float32),
            pltpu.VMEM((_NBUF, _BM, d), jnp.float32),
            pltpu.VMEM((d, d), jnp.bfloat16),
            pltpu.SemaphoreType.DMA((_NBUF,)),
            pltpu.SemaphoreType.DMA((_NBUF,)),
            pltpu.SemaphoreType.DMA,
        ],
    )(flat, W, b.reshape(1, d))
